# baseline (speedup 1.0000x reference)
import functools

import jax
import jax.numpy as jnp
from jax import lax
from jax.experimental import pallas as pl
from jax.experimental.pallas import tpu as pltpu
from jax.experimental.pallas import tpu_sc as plsc

_EPS = 1e-12

_NC = 2
_NS = 16
_NW = _NC * _NS

_NCHUNK = 4


def _sc_gather(table, idx):
    BS = idx.shape[0]
    H = table.shape[1]
    b_per_w = BS // _NW
    CH = 40
    n_ch = b_per_w // CH
    n2 = n_ch // 2

    mesh = plsc.VectorSubcoreMesh(core_axis_name="c", subcore_axis_name="s")

    @functools.partial(
        pl.kernel,
        mesh=mesh,
        out_type=jax.ShapeDtypeStruct((BS, H), jnp.float32),
        scratch_types=[
            pltpu.VMEM((b_per_w,), jnp.int32),
            pltpu.VMEM((CH, H), jnp.float32),
            pltpu.VMEM((CH, H), jnp.float32),
            pltpu.SemaphoreType.DMA,
            pltpu.SemaphoreType.DMA,
            pltpu.SemaphoreType.DMA,
            pltpu.SemaphoreType.DMA,
        ],
    )
    def k(idx_hbm, table_hbm, out_hbm, idx_v, buf0, buf1, gs0, gs1, ws0, ws1):
        wid = lax.axis_index("s") * _NC + lax.axis_index("c")
        base = wid * b_per_w
        pltpu.sync_copy(idx_hbm.at[pl.ds(base, b_per_w)], idx_v)

        def g_start(c, buf, sem):
            pltpu.async_copy(table_hbm.at[idx_v.at[pl.ds(c * CH, CH)]],
                             buf, sem)

        def g_wait(buf, sem):
            pltpu.make_async_copy(table_hbm.at[idx_v.at[pl.ds(0, CH)]],
                                  buf, sem).wait()

        def w_start(c, buf, sem):
            pltpu.async_copy(buf, out_hbm.at[pl.ds(base + c * CH, CH)], sem)

        def w_wait(buf, sem):
            pltpu.make_async_copy(buf, out_hbm.at[pl.ds(base, CH)],
                                  sem).wait()

        g_start(0, buf0, gs0)

        def body(jj, carry):
            c0 = 2 * jj
            g_wait(buf0, gs0)
            g_start(c0 + 1, buf1, gs1)
            w_start(c0, buf0, ws0)
            g_wait(buf1, gs1)
            w_wait(buf0, ws0)
            g_start(c0 + 2, buf0, gs0)
            w_start(c0 + 1, buf1, ws1)
            w_wait(buf1, ws1)
            return carry

        lax.fori_loop(0, n2 - 1, body, 0)

        c0 = n_ch - 2
        g_wait(buf0, gs0)
        g_start(c0 + 1, buf1, gs1)
        w_start(c0, buf0, ws0)
        g_wait(buf1, gs1)
        w_wait(buf0, ws0)
        w_start(c0 + 1, buf1, ws1)
        w_wait(buf1, ws1)

    return k(idx, table)


def _tc_chunk(prev_out, gathered, ttf, ve3, vttf, pos_s, tte, vtte, vpos0,
              wt, b2, gam, bet, chunk, n_steps, BB, B, S, V, H, VD):
    c0 = chunk * n_steps

    def body(po_ref, g_ref, tt_ref, ve_ref, vtt_ref, pos_ref, tte_ref,
             vtte_ref, vpos_ref, w_ref, b_ref, gam_ref, bet_ref, o_ref):
        g = gam_ref[0, :]
        be = bet_ref[0, :]

        def ln(x):
            mu = jnp.mean(x, axis=-1, keepdims=True)
            xc = x - mu
            var = jnp.mean(xc * xc, axis=-1, keepdims=True)
            return xc * lax.rsqrt(var + _EPS) * g + be

        t0 = tte_ref[0, :]
        t1 = tte_ref[1, :]
        v0 = vtte_ref[0, :] + vpos_ref[0, :] + b_ref[0, :]
        v1 = vtte_ref[1, :] + vpos_ref[0, :] + b_ref[0, :]
        pbase = pos_ref[...] + t0[None, :]
        tdiff = (t1 - t0)[None, None, :]
        g3 = g_ref[...].reshape(BB, S, H)
        tt3 = tt_ref[...][:, :, None]
        text = g3 + pbase[None, :, :] + tt3 * tdiff
        o_ref[:, :S, :] = ln(text)
        veb = ve_ref[...].astype(jnp.bfloat16)
        vm = lax.dot_general(veb, w_ref[...], (((1,), (1,)), ((), ())),
                             preferred_element_type=jnp.float32)
        vtt2 = vtt_ref[...]
        vis = ln(vm + v0[None, :] + vtt2 * (v1 - v0)[None, :])
        o_ref[:, S:, :] = vis.reshape(BB, V, H)

    body_fn = body
    if prev_out is None:
        def body_fn(*refs):
            body(None, *refs)

    in_specs = [
        pl.BlockSpec((BB * S, H), lambda i: (i, 0)),
        pl.BlockSpec((BB, S), lambda i: (c0 + i, 0)),
        pl.BlockSpec((BB * V, VD), lambda i: (c0 + i, 0)),
        pl.BlockSpec((BB * V, 1), lambda i: (c0 + i, 0)),
        pl.BlockSpec((S, H), lambda i: (0, 0)),
        pl.BlockSpec((2, H), lambda i: (0, 0)),
        pl.BlockSpec((2, H), lambda i: (0, 0)),
        pl.BlockSpec((1, H), lambda i: (0, 0)),
        pl.BlockSpec((H, VD), lambda i: (0, 0)),
        pl.BlockSpec((1, H), lambda i: (0, 0)),
        pl.BlockSpec((1, H), lambda i: (0, 0)),
        pl.BlockSpec((1, H), lambda i: (0, 0)),
    ]
    args = [gathered, ttf, ve3, vttf, pos_s, tte, vtte, vpos0,
            wt, b2, gam, bet]
    aliases = {}
    if prev_out is not None:
        in_specs = [pl.BlockSpec(memory_space=pl.ANY)] + in_specs
        args = [prev_out] + args
        aliases = {0: 0}

    return pl.pallas_call(
        body_fn if prev_out is None else body,
        grid=(n_steps,),
        in_specs=in_specs,
        out_specs=pl.BlockSpec((BB, S + V, H), lambda i: (c0 + i, 0, 0)),
        out_shape=jax.ShapeDtypeStruct((B, S + V, H), jnp.float32),
        input_output_aliases=aliases,
    )(*args)


def kernel(input_ids, token_type_ids, visual_embeds, visual_token_type_ids,
           word_emb, pos_emb, tok_type_emb, vis_tok_type_emb, vis_pos_emb,
           vproj_w, vproj_b, ln_gamma, ln_beta):
    B, S = input_ids.shape
    H = word_emb.shape[1]
    V, VD = visual_embeds.shape[1], visual_embeds.shape[2]
    BB = 8
    Bc = B // _NCHUNK
    n_steps = Bc // BB

    ids = input_ids.reshape(-1)
    ttf = token_type_ids.astype(jnp.float32)
    vttf = visual_token_type_ids.astype(jnp.float32).reshape(B * V, 1)
    ve2 = visual_embeds.reshape(B * V, VD)
    wt = vproj_w.astype(jnp.bfloat16)
    common = (ttf, ve2, vttf, pos_emb[:S], tok_type_emb,
              vis_tok_type_emb, vis_pos_emb[0:1], wt, vproj_b.reshape(1, H),
              ln_gamma.reshape(1, H), ln_beta.reshape(1, H))

    gs = [_sc_gather(word_emb, ids[c * Bc * S:(c + 1) * Bc * S])
          for c in range(_NCHUNK)]
    out = None
    for c in range(_NCHUNK):
        out = _tc_chunk(out, gs[c], *common, chunk=c, n_steps=n_steps,
                        BB=BB, B=B, S=S, V=V, H=H, VD=VD)
    return out


# speedup vs baseline: 1.0067x; 1.0067x over previous
import functools

import jax
import jax.numpy as jnp
from jax import lax
from jax.experimental import pallas as pl
from jax.experimental.pallas import tpu as pltpu
from jax.experimental.pallas import tpu_sc as plsc

_EPS = 1e-12

_NC = 2
_NS = 16
_NW = _NC * _NS

_NCHUNK = 2


def _sc_gather(table, idx):
    BS = idx.shape[0]
    H = table.shape[1]
    b_per_w = BS // _NW
    CH = 80
    n_ch = b_per_w // CH
    n2 = n_ch // 2

    mesh = plsc.VectorSubcoreMesh(core_axis_name="c", subcore_axis_name="s")

    @functools.partial(
        pl.kernel,
        mesh=mesh,
        out_type=jax.ShapeDtypeStruct((BS, H), jnp.float32),
        scratch_types=[
            pltpu.VMEM((b_per_w,), jnp.int32),
            pltpu.VMEM((CH, H), jnp.float32),
            pltpu.VMEM((CH, H), jnp.float32),
            pltpu.SemaphoreType.DMA,
            pltpu.SemaphoreType.DMA,
            pltpu.SemaphoreType.DMA,
            pltpu.SemaphoreType.DMA,
        ],
    )
    def k(idx_hbm, table_hbm, out_hbm, idx_v, buf0, buf1, gs0, gs1, ws0, ws1):
        wid = lax.axis_index("s") * _NC + lax.axis_index("c")
        base = wid * b_per_w
        pltpu.sync_copy(idx_hbm.at[pl.ds(base, b_per_w)], idx_v)

        def g_start(c, buf, sem):
            pltpu.async_copy(table_hbm.at[idx_v.at[pl.ds(c * CH, CH)]],
                             buf, sem)

        def g_wait(buf, sem):
            pltpu.make_async_copy(table_hbm.at[idx_v.at[pl.ds(0, CH)]],
                                  buf, sem).wait()

        def w_start(c, buf, sem):
            pltpu.async_copy(buf, out_hbm.at[pl.ds(base + c * CH, CH)], sem)

        def w_wait(buf, sem):
            pltpu.make_async_copy(buf, out_hbm.at[pl.ds(base, CH)],
                                  sem).wait()

        g_start(0, buf0, gs0)

        def body(jj, carry):
            c0 = 2 * jj
            g_wait(buf0, gs0)
            g_start(c0 + 1, buf1, gs1)
            w_start(c0, buf0, ws0)
            g_wait(buf1, gs1)
            w_wait(buf0, ws0)
            g_start(c0 + 2, buf0, gs0)
            w_start(c0 + 1, buf1, ws1)
            w_wait(buf1, ws1)
            return carry

        lax.fori_loop(0, n2 - 1, body, 0)

        c0 = n_ch - 2
        g_wait(buf0, gs0)
        g_start(c0 + 1, buf1, gs1)
        w_start(c0, buf0, ws0)
        g_wait(buf1, gs1)
        w_wait(buf0, ws0)
        w_start(c0 + 1, buf1, ws1)
        w_wait(buf1, ws1)

    return k(idx, table)


def _tc_chunk(prev_out, gathered, ttf, ve3, vttf, pos_s, tte, vtte, vpos0,
              wt, b2, gam, bet, chunk, n_steps, BB, B, S, V, H, VD):
    c0 = chunk * n_steps

    def body(po_ref, g_ref, tt_ref, ve_ref, vtt_ref, pos_ref, tte_ref,
             vtte_ref, vpos_ref, w_ref, b_ref, gam_ref, bet_ref, o_ref):
        g = gam_ref[0, :]
        be = bet_ref[0, :]

        def ln(x):
            mu = jnp.mean(x, axis=-1, keepdims=True)
            xc = x - mu
            var = jnp.mean(xc * xc, axis=-1, keepdims=True)
            return xc * lax.rsqrt(var + _EPS) * g + be

        t0 = tte_ref[0, :]
        t1 = tte_ref[1, :]
        v0 = vtte_ref[0, :] + vpos_ref[0, :] + b_ref[0, :]
        v1 = vtte_ref[1, :] + vpos_ref[0, :] + b_ref[0, :]
        pbase = pos_ref[...] + t0[None, :]
        tdiff = (t1 - t0)[None, None, :]
        g3 = g_ref[...].reshape(BB, S, H)
        tt3 = tt_ref[...][:, :, None]
        text = g3 + pbase[None, :, :] + tt3 * tdiff
        o_ref[:, :S, :] = ln(text)
        veb = ve_ref[...].astype(jnp.bfloat16)
        vm = lax.dot_general(veb, w_ref[...], (((1,), (1,)), ((), ())),
                             preferred_element_type=jnp.float32)
        vtt2 = vtt_ref[...]
        vis = ln(vm + v0[None, :] + vtt2 * (v1 - v0)[None, :])
        o_ref[:, S:, :] = vis.reshape(BB, V, H)

    body_fn = body
    if prev_out is None:
        def body_fn(*refs):
            body(None, *refs)

    in_specs = [
        pl.BlockSpec((BB * S, H), lambda i: (i, 0)),
        pl.BlockSpec((BB, S), lambda i: (c0 + i, 0)),
        pl.BlockSpec((BB * V, VD), lambda i: (c0 + i, 0)),
        pl.BlockSpec((BB * V, 1), lambda i: (c0 + i, 0)),
        pl.BlockSpec((S, H), lambda i: (0, 0)),
        pl.BlockSpec((2, H), lambda i: (0, 0)),
        pl.BlockSpec((2, H), lambda i: (0, 0)),
        pl.BlockSpec((1, H), lambda i: (0, 0)),
        pl.BlockSpec((H, VD), lambda i: (0, 0)),
        pl.BlockSpec((1, H), lambda i: (0, 0)),
        pl.BlockSpec((1, H), lambda i: (0, 0)),
        pl.BlockSpec((1, H), lambda i: (0, 0)),
    ]
    args = [gathered, ttf, ve3, vttf, pos_s, tte, vtte, vpos0,
            wt, b2, gam, bet]
    aliases = {}
    if prev_out is not None:
        in_specs = [pl.BlockSpec(memory_space=pl.ANY)] + in_specs
        args = [prev_out] + args
        aliases = {0: 0}

    return pl.pallas_call(
        body_fn if prev_out is None else body,
        grid=(n_steps,),
        in_specs=in_specs,
        out_specs=pl.BlockSpec((BB, S + V, H), lambda i: (c0 + i, 0, 0)),
        out_shape=jax.ShapeDtypeStruct((B, S + V, H), jnp.float32),
        input_output_aliases=aliases,
    )(*args)


def kernel(input_ids, token_type_ids, visual_embeds, visual_token_type_ids,
           word_emb, pos_emb, tok_type_emb, vis_tok_type_emb, vis_pos_emb,
           vproj_w, vproj_b, ln_gamma, ln_beta):
    B, S = input_ids.shape
    H = word_emb.shape[1]
    V, VD = visual_embeds.shape[1], visual_embeds.shape[2]
    BB = 8
    Bc = B // _NCHUNK
    n_steps = Bc // BB

    ids = input_ids.reshape(-1)
    ttf = token_type_ids.astype(jnp.float32)
    vttf = visual_token_type_ids.astype(jnp.float32).reshape(B * V, 1)
    ve2 = visual_embeds.reshape(B * V, VD)
    wt = vproj_w.astype(jnp.bfloat16)
    common = (ttf, ve2, vttf, pos_emb[:S], tok_type_emb,
              vis_tok_type_emb, vis_pos_emb[0:1], wt, vproj_b.reshape(1, H),
              ln_gamma.reshape(1, H), ln_beta.reshape(1, H))

    gs = [_sc_gather(word_emb, ids[c * Bc * S:(c + 1) * Bc * S])
          for c in range(_NCHUNK)]
    out = None
    for c in range(_NCHUNK):
        out = _tc_chunk(out, gs[c], *common, chunk=c, n_steps=n_steps,
                        BB=BB, B=B, S=S, V=V, H=H, VD=VD)
    return out
